# baseline (speedup 1.0000x reference)
import functools

import jax
import jax.numpy as jnp
from jax import lax
from jax.experimental import pallas as pl
from jax.experimental.pallas import tpu as pltpu
from jax.experimental.pallas import tpu_sc as plsc

NBUF = 2


@functools.lru_cache(maxsize=None)
def _make_pool(B, H, V, D):
    NC, NS, L = 2, 16, 16
    NW = NC * NS
    assert B % NW == 0
    bpw = B // NW
    assert H % 2 == 0 and H // 2 <= 128
    ch = H // 2
    assert D % L == 0
    nv = D // L

    mesh = plsc.VectorSubcoreMesh(core_axis_name="c", subcore_axis_name="s")

    @functools.partial(
        pl.kernel,
        mesh=mesh,
        compiler_params=pltpu.CompilerParams(use_tc_tiling_on_sc=False),
        out_type=jax.ShapeDtypeStruct((B, D), jnp.float32),
        scratch_types=[
            pltpu.VMEM((bpw, 2, ch), jnp.int32),
            pltpu.VMEM((NBUF, H, D), jnp.float32),
            pltpu.VMEM((bpw, D), jnp.float32),
            pltpu.SemaphoreType.DMA((NBUF,)),
        ],
    )
    def pool(x_hbm, table_hbm, out_hbm, idx_v, rows_v, out_v, sems):
        wid = lax.axis_index("s") * NC + lax.axis_index("c")
        base = wid * bpw
        pltpu.sync_copy(x_hbm.at[pl.ds(base, bpw)], idx_v)

        def issue(item, p):
            for h in range(2):
                pltpu.async_copy(
                    table_hbm.at[idx_v.at[item, h]],
                    rows_v.at[p, pl.ds(h * ch, ch)],
                    sems.at[p],
                )

        def drain(p):
            pltpu.make_async_copy(
                table_hbm.at[pl.ds(0, H)], rows_v.at[p], sems.at[p]
            ).wait()

        for p in range(NBUF):
            issue(p, p)

        inv = jnp.float32(1.0 / H)
        zero = jnp.zeros((L,), jnp.float32)

        def outer(g, carry):
            for p in range(NBUF):
                i = g * NBUF + p
                drain(p)

                def body(j, accs):
                    return tuple(
                        accs[k] + rows_v[p, j, pl.ds(k * L, L)]
                        for k in range(nv)
                    )

                accs = lax.fori_loop(0, H, body, (zero,) * nv)

                nxt = i + NBUF

                @pl.when(nxt < bpw)
                def _():
                    issue(nxt, p)

                for k in range(nv):
                    out_v[i, pl.ds(k * L, L)] = accs[k] * inv
            return carry

        lax.fori_loop(0, bpw // NBUF, outer, 0)
        pltpu.sync_copy(out_v, out_hbm.at[pl.ds(base, bpw)])

    return pool


def _linear_body(p_ref, wt_ref, b_ref, o_ref):
    o_ref[...] = (
        jnp.dot(p_ref[...], wt_ref[...], preferred_element_type=jnp.float32)
        + b_ref[...]
    )


@functools.lru_cache(maxsize=None)
def _make_linear(B, D, O):
    return pl.pallas_call(
        _linear_body,
        out_shape=jax.ShapeDtypeStruct((B, O), jnp.float32),
    )


def kernel(x, table, W, b):
    B, H = x.shape
    V, D = table.shape
    O = W.shape[0]
    x3 = x.astype(jnp.int32).reshape(B, 2, H // 2)
    pooled = _make_pool(B, H, V, D)(x3, table)
    return _make_linear(B, D, O)(pooled, W.T, b[None, :])
